# baseline (speedup 1.0000x reference)
import jax
import jax.numpy as jnp
from jax.experimental import pallas as pl
from jax.experimental.pallas import tpu as pltpu

H = 512
W = 512


G = 16


def _mask_kernel(bounds_ref, o_ref):
    g = pl.program_id(0)
    iy = jax.lax.broadcasted_iota(jnp.int32, (H, 1), 0)
    ix = jax.lax.broadcasted_iota(jnp.int32, (1, W), 1)
    for i in range(G):
        b = g * G + i
        xs = bounds_ref[b, 0]
        xe = bounds_ref[b, 1]
        ys = bounds_ref[b, 2]
        ye = bounds_ref[b, 3]
        row_f = ((iy >= ys) & (iy <= ye)).astype(jnp.float32)
        col_f = ((ix >= xs) & (ix <= xe)).astype(jnp.float32)
        o_ref[i] = row_f * col_f


def kernel(motion):
    B = motion.shape[0]
    xs = jnp.ceil(jnp.float32(H // 4) + motion[:, 0]).astype(jnp.int32)
    xe = jnp.ceil(jnp.float32(3 * H // 4) + motion[:, 0]).astype(jnp.int32)
    ys = jnp.ceil(jnp.float32(W // 4) + motion[:, 1]).astype(jnp.int32)
    ye = jnp.ceil(jnp.float32(3 * W // 4) + motion[:, 1]).astype(jnp.int32)
    bounds = jnp.stack([xs, xe, ys, ye], axis=1)

    out = pl.pallas_call(
        _mask_kernel,
        grid=(B // G,),
        in_specs=[pl.BlockSpec(memory_space=pltpu.SMEM)],
        out_specs=pl.BlockSpec((G, H, W), lambda g: (g, 0, 0)),
        out_shape=jax.ShapeDtypeStruct((B, H, W), jnp.float32),
    )(bounds)
    return out[..., None]


# speedup vs baseline: 1.6137x; 1.6137x over previous
import functools

import jax
import jax.numpy as jnp
from jax import lax
from jax.experimental import pallas as pl
from jax.experimental.pallas import tpu as pltpu
from jax.experimental.pallas import tpu_sc as plsc

H = 512
W = 512
LANES = 16
ROWS_PER_BLOCK = 64
BLOCK = ROWS_PER_BLOCK * W
NC = 2
NS = 16
NW = NC * NS


def _lane(vec, j):
    return vec[j]


def _span_dmas(src_ref, out_ref, s, e):
    n = e - s
    big = n >= ROWS_PER_BLOCK
    k_chunk = jnp.where(big, (n + ROWS_PER_BLOCK - 1) // ROWS_PER_BLOCK, 0)
    k_row = jnp.where(big, 0, jnp.maximum(n, 0))

    def chunk_body(i, carry):
        start = jnp.minimum(s + i * ROWS_PER_BLOCK, e - ROWS_PER_BLOCK)
        pltpu.sync_copy(src_ref.at[pl.ds(0, BLOCK)],
                        out_ref.at[pl.ds(start * W, BLOCK)])
        return carry

    def row_body(i, carry):
        pltpu.sync_copy(src_ref.at[pl.ds(0, W)],
                        out_ref.at[pl.ds((s + i) * W, W)])
        return carry

    lax.fori_loop(0, k_chunk, chunk_body, 0)
    lax.fori_loop(0, k_row, row_body, 0)


def _sc_kernel(bounds_hbm, out_hbm, pat_ref, zero_ref, bvec_ref):
    wid = lax.axis_index("s") * NC + lax.axis_index("c")

    zv = jnp.zeros((LANES,), jnp.float32)

    def zrow(r, carry):
        for c in range(W // LANES):
            zero_ref[pl.ds(r * W + c * LANES, LANES)] = zv
        return carry

    lax.fori_loop(0, ROWS_PER_BLOCK, zrow, 0)

    for t in range(2):
        b = wid * 2 + t
        pltpu.sync_copy(bounds_hbm.at[b], bvec_ref)
        bv = bvec_ref[...]
        xs = _lane(bv, 0)
        xe = _lane(bv, 1)
        ys = _lane(bv, 2)
        ye = _lane(bv, 3)
        lo = jnp.clip(ys, 0, H)
        hi = jnp.clip(ye + 1, lo, H)

        def prow(r, carry):
            for c in range(W // LANES):
                ix = lax.broadcasted_iota(jnp.int32, (LANES,), 0) + c * LANES
                val = jnp.where((ix >= xs) & (ix <= xe),
                                jnp.float32(1.0), jnp.float32(0.0))
                pat_ref[pl.ds(r * W + c * LANES, LANES)] = val
            return carry

        lax.fori_loop(0, ROWS_PER_BLOCK, prow, 0)

        base = b * H
        _span_dmas(zero_ref, out_hbm, base, base + lo)
        _span_dmas(pat_ref, out_hbm, base + lo, base + hi)
        _span_dmas(zero_ref, out_hbm, base + hi, base + H)


def kernel(motion):
    B = motion.shape[0]
    xs = jnp.ceil(jnp.float32(H // 4) + motion[:, 0]).astype(jnp.int32)
    xe = jnp.ceil(jnp.float32(3 * H // 4) + motion[:, 0]).astype(jnp.int32)
    ys = jnp.ceil(jnp.float32(W // 4) + motion[:, 1]).astype(jnp.int32)
    ye = jnp.ceil(jnp.float32(3 * W // 4) + motion[:, 1]).astype(jnp.int32)
    bounds = jnp.zeros((B, LANES), jnp.int32)
    bounds = bounds.at[:, 0].set(xs).at[:, 1].set(xe)
    bounds = bounds.at[:, 2].set(ys).at[:, 3].set(ye)

    run = functools.partial(
        pl.kernel,
        mesh=plsc.VectorSubcoreMesh(core_axis_name="c", subcore_axis_name="s"),
        out_type=jax.ShapeDtypeStruct((B * H * W,), jnp.float32),
        scratch_types=[
            pltpu.VMEM((BLOCK,), jnp.float32),
            pltpu.VMEM((BLOCK,), jnp.float32),
            pltpu.VMEM((LANES,), jnp.int32),
        ],
    )(_sc_kernel)
    out = run(bounds)
    return out.reshape(B, H, W, 1)


# speedup vs baseline: 1.6636x; 1.0309x over previous
import functools

import jax
import jax.numpy as jnp
from jax import lax
from jax.experimental import pallas as pl
from jax.experimental.pallas import tpu as pltpu
from jax.experimental.pallas import tpu_sc as plsc

H = 512
W = 512
LANES = 16
ROWS_PER_BLOCK = 64
BLOCK = ROWS_PER_BLOCK * W
NC = 2
NS = 16
NW = NC * NS


def _lane(vec, j):
    return vec[j]


def _span_dmas(src_ref, out_ref, s, e, sem, do_start):
    n = e - s
    big = n >= ROWS_PER_BLOCK
    k_chunk = jnp.where(big, (n + ROWS_PER_BLOCK - 1) // ROWS_PER_BLOCK, 0)
    k_row = jnp.where(big, 0, jnp.maximum(n, 0))

    def chunk_body(i, carry):
        start = jnp.minimum(s + i * ROWS_PER_BLOCK, e - ROWS_PER_BLOCK)
        cp = pltpu.make_async_copy(src_ref.at[pl.ds(0, BLOCK)],
                                   out_ref.at[pl.ds(start * W, BLOCK)], sem)
        if do_start:
            cp.start()
        else:
            cp.wait()
        return carry

    def row_body(i, carry):
        cp = pltpu.make_async_copy(src_ref.at[pl.ds(0, W)],
                                   out_ref.at[pl.ds((s + i) * W, W)], sem)
        if do_start:
            cp.start()
        else:
            cp.wait()
        return carry

    lax.fori_loop(0, k_chunk, chunk_body, 0)
    lax.fori_loop(0, k_row, row_body, 0)


def _sc_kernel(bounds_hbm, out_hbm, pat_refs, zero_ref, bvec_ref, sem):
    wid = lax.axis_index("s") * NC + lax.axis_index("c")

    zv = jnp.zeros((LANES,), jnp.float32)

    def zrow(r, carry):
        for c in range(W // LANES):
            zero_ref[pl.ds(r * W + c * LANES, LANES)] = zv
        return carry

    lax.fori_loop(0, ROWS_PER_BLOCK, zrow, 0)

    spans = []
    for t in range(2):
        b = wid * 2 + t
        pat_ref = pat_refs[t]
        pltpu.sync_copy(bounds_hbm.at[b], bvec_ref)
        bv = bvec_ref[...]
        xs = _lane(bv, 0)
        xe = _lane(bv, 1)
        ys = _lane(bv, 2)
        ye = _lane(bv, 3)
        lo = jnp.clip(ys, 0, H)
        hi = jnp.clip(ye + 1, lo, H)

        def prow(r, carry):
            for c in range(W // LANES):
                ix = lax.broadcasted_iota(jnp.int32, (LANES,), 0) + c * LANES
                val = jnp.where((ix >= xs) & (ix <= xe),
                                jnp.float32(1.0), jnp.float32(0.0))
                pat_ref[pl.ds(r * W + c * LANES, LANES)] = val
            return carry

        lax.fori_loop(0, ROWS_PER_BLOCK, prow, 0)

        base = b * H
        spans += [(zero_ref, base, base + lo),
                  (pat_ref, base + lo, base + hi),
                  (zero_ref, base + hi, base + H)]
        for ref, s0, s1 in spans[-3:]:
            _span_dmas(ref, out_hbm, s0, s1, sem, do_start=True)

    for ref, s0, s1 in spans:
        _span_dmas(ref, out_hbm, s0, s1, sem, do_start=False)


def kernel(motion):
    B = motion.shape[0]
    xs = jnp.ceil(jnp.float32(H // 4) + motion[:, 0]).astype(jnp.int32)
    xe = jnp.ceil(jnp.float32(3 * H // 4) + motion[:, 0]).astype(jnp.int32)
    ys = jnp.ceil(jnp.float32(W // 4) + motion[:, 1]).astype(jnp.int32)
    ye = jnp.ceil(jnp.float32(3 * W // 4) + motion[:, 1]).astype(jnp.int32)
    bounds = jnp.zeros((B, LANES), jnp.int32)
    bounds = bounds.at[:, 0].set(xs).at[:, 1].set(xe)
    bounds = bounds.at[:, 2].set(ys).at[:, 3].set(ye)

    run = functools.partial(
        pl.kernel,
        mesh=plsc.VectorSubcoreMesh(core_axis_name="c", subcore_axis_name="s"),
        out_type=jax.ShapeDtypeStruct((B * H * W,), jnp.float32),
        scratch_types=[
            (pltpu.VMEM((BLOCK,), jnp.float32),
             pltpu.VMEM((BLOCK,), jnp.float32)),
            pltpu.VMEM((BLOCK,), jnp.float32),
            pltpu.VMEM((LANES,), jnp.int32),
            pltpu.SemaphoreType.DMA,
        ],
    )(_sc_kernel)
    out = run(bounds)
    return out.reshape(B, H, W, 1)
